# baseline (speedup 1.0000x reference)
import functools

import jax
import jax.numpy as jnp
from jax import lax
from jax.experimental import pallas as pl
from jax.experimental.pallas import tpu as pltpu
from jax.experimental.pallas import tpu_sc as plsc

_NC = 2
_NS = 16
_NW = _NC * _NS
_LANES = 16
_IDX_CHUNK = 128
_PACK = 4


def _make_sc_kernel(batch, dim):
    assert batch % (8 * _NW) == 0
    assert dim == 2 * _LANES
    b_per_w = batch // _NW
    half = b_per_w // 2
    mesh = plsc.VectorSubcoreMesh(core_axis_name="c", subcore_axis_name="s")
    row_w = dim * _PACK

    @functools.partial(
        pl.kernel,
        mesh=mesh,
        compiler_params=pltpu.CompilerParams(needs_layout_passes=False),
        out_type=jax.ShapeDtypeStruct((batch,), jnp.float32),
        scratch_types=[
            pltpu.VMEM((b_per_w,), jnp.int32),
            pltpu.VMEM((b_per_w,), jnp.int32),
            pltpu.VMEM((b_per_w,), jnp.int32),
            pltpu.VMEM((b_per_w,), jnp.int32),
            pltpu.VMEM((half, row_w), jnp.float32),
            pltpu.VMEM((half, row_w), jnp.float32),
            pltpu.VMEM((b_per_w,), jnp.float32),
            pltpu.SemaphoreType.DMA,
            pltpu.SemaphoreType.DMA,
        ],
    )
    def sc_kernel(uid_hbm, iid_hbm, utab_hbm, itab_hbm, out_hbm,
                  uidx_v, iidx_v, uq_v, iq_v, urows_v, irows_v, out_v,
                  usem, isem):
        wid = lax.axis_index("s") * _NC + lax.axis_index("c")
        base = wid * b_per_w

        pltpu.sync_copy(uid_hbm.at[pl.ds(base, b_per_w)], uidx_v)
        pltpu.sync_copy(iid_hbm.at[pl.ds(base, b_per_w)], iidx_v)
        for t in range(b_per_w // _LANES):
            sl = pl.ds(t * _LANES, _LANES)
            uq_v[sl] = jax.lax.shift_right_logical(uidx_v[sl], 2)
            iq_v[sl] = jax.lax.shift_right_logical(iidx_v[sl], 2)

        lane_iota = lax.iota(jnp.int32, _LANES)

        for h in range(2):
            for j in range(half // _IDX_CHUNK):
                isl = pl.ds(h * half + j * _IDX_CHUNK, _IDX_CHUNK)
                dsl = pl.ds(j * _IDX_CHUNK, _IDX_CHUNK)
                pltpu.async_copy(utab_hbm.at[uq_v.at[isl]],
                                 urows_v.at[dsl], usem)
                pltpu.async_copy(itab_hbm.at[iq_v.at[isl]],
                                 irows_v.at[dsl], isem)
            pltpu.make_async_copy(utab_hbm.at[pl.ds(0, half)], urows_v,
                                  usem).wait()
            pltpu.make_async_copy(itab_hbm.at[pl.ds(0, half)], irows_v,
                                  isem).wait()

            def body(g, _):
                off = g * _LANES
                rvec = off + lane_iota
                ucol = (uidx_v[pl.ds(h * half + off, _LANES)] & 3) * dim
                icol = (iidx_v[pl.ds(h * half + off, _LANES)] & 3) * dim
                acc = jnp.zeros((_LANES,), jnp.float32)
                for k in range(dim):
                    u = plsc.load_gather(urows_v, [rvec, ucol + k])
                    i = plsc.load_gather(irows_v, [rvec, icol + k])
                    acc = acc + u * i
                out_v[pl.ds(h * half + off, _LANES)] = acc
                return 0

            lax.fori_loop(0, half // _LANES, body, 0)

        pltpu.sync_copy(out_v, out_hbm.at[pl.ds(base, b_per_w)])

    return sc_kernel


@jax.jit
def kernel(user_id, item_id, user_table, item_table):
    batch = user_id.shape[0]
    rows, dim = user_table.shape
    fn = _make_sc_kernel(batch, dim)
    ulin = user_table.reshape(rows // _PACK, dim * _PACK)
    ilin = item_table.reshape(rows // _PACK, dim * _PACK)
    return fn(user_id, item_id, ulin, ilin)
